# baseline (speedup 1.0000x reference)
import functools

import jax
import jax.numpy as jnp
from jax import lax
from jax.experimental import pallas as pl
from jax.experimental.pallas import tpu as pltpu
from jax.experimental.pallas import tpu_sc as plsc

_LANES = 128
_NBUF = 2
_BR = 8
_C0 = 96 * 1024
_SC_CH = 1024


def _tc_argmax_body(x_hbm, max_ref, idx_ref, buf, sem):
    b, v = x_hbm.shape
    w = v - _C0
    copy = pltpu.make_async_copy(x_hbm.at[:, pl.ds(_C0, w)], buf, sem)
    copy.start()
    copy.wait()
    x = buf[...]
    m = jnp.max(x, axis=1, keepdims=True)
    ii = lax.broadcasted_iota(jnp.int32, x.shape, 1) + _C0
    cand = jnp.where(x == m, ii, jnp.int32(10**9))
    idx_ref[...] = jnp.min(cand, axis=1, keepdims=True)
    max_ref[...] = m


def _argmax_tc(inputs):
    b, v = inputs.shape
    w = v - _C0
    return pl.pallas_call(
        _tc_argmax_body,
        in_specs=[pl.BlockSpec(memory_space=pl.ANY)],
        out_specs=[
            pl.BlockSpec(memory_space=pltpu.MemorySpace.VMEM),
            pl.BlockSpec(memory_space=pltpu.MemorySpace.VMEM),
        ],
        out_shape=[
            jax.ShapeDtypeStruct((b, 1), jnp.float32),
            jax.ShapeDtypeStruct((b, 1), jnp.int32),
        ],
        scratch_shapes=[
            pltpu.VMEM((b, w), jnp.float32),
            pltpu.SemaphoreType.DMA,
        ],
        compiler_params=pltpu.CompilerParams(vmem_limit_bytes=100 * 1024 * 1024),
    )(inputs)


def _argmax_sc(inputs):
    b, v = inputs.shape
    info = plsc.get_sparse_core_info()
    nw = info.num_cores * info.num_subcores
    nl = info.num_lanes
    rows_per_w = b // nw
    ngroups = rows_per_w // 8
    nch = _C0 // _SC_CH
    mesh = plsc.VectorSubcoreMesh(core_axis_name="c", subcore_axis_name="s", num_cores=2)

    @functools.partial(
        pl.kernel,
        mesh=mesh,
        out_type=(
            jax.ShapeDtypeStruct((b,), jnp.float32),
            jax.ShapeDtypeStruct((b,), jnp.int32),
        ),
        scratch_types=[
            pltpu.VMEM((_NBUF, 8, _SC_CH), jnp.float32),
            pltpu.VMEM((rows_per_w,), jnp.float32),
            pltpu.VMEM((rows_per_w,), jnp.int32),
            pltpu.SemaphoreType.DMA((_NBUF,)),
        ],
        compiler_params=pltpu.CompilerParams(needs_layout_passes=False),
    )
    def sc_argmax_kernel(x_hbm, max_hbm, idx_hbm, buf, mstage, istage, sems):
        wid = lax.axis_index("s") * info.num_cores + lax.axis_index("c")
        row0 = wid * rows_per_w
        lane = lax.iota(jnp.int32, nl)
        nvec = _SC_CH // nl

        def start(par, grow, ch):
            pltpu.make_async_copy(
                x_hbm.at[pl.ds(grow, 8), pl.ds(ch * _SC_CH, _SC_CH)],
                buf.at[par],
                sems.at[par],
            ).start()

        def wait(par):
            pltpu.make_async_copy(
                x_hbm.at[pl.ds(0, 8), pl.ds(0, _SC_CH)],
                buf.at[par],
                sems.at[par],
            ).wait()

        for g in range(ngroups):
            grow = row0 + g * 8
            start(0, grow, 0)
            start(1, grow, 1)
            neg = jnp.full((nl,), -jnp.inf, jnp.float32)
            zero = jnp.zeros((nl,), jnp.int32)
            carry0 = (neg,) * 8 + (zero,) * 8

            def dbl_step(i, carry):
                for par in range(_NBUF):
                    ch = i * _NBUF + par
                    wait(par)

                    def vec_step(j, carry2):
                        ms = carry2[:8]
                        gs = carry2[8:]
                        giota = lane + (ch * _SC_CH + j * nl)
                        new_m = []
                        new_g = []
                        for r in range(8):
                            chunk = buf[par, r, pl.ds(j * nl, nl)]
                            upd = chunk > ms[r]
                            new_m.append(jnp.where(upd, chunk, ms[r]))
                            new_g.append(jnp.where(upd, giota, gs[r]))
                        return tuple(new_m) + tuple(new_g)

                    carry = lax.fori_loop(0, nvec, vec_step, carry)

                    @pl.when(ch + _NBUF < nch)
                    def _():
                        start(par, grow, ch + _NBUF)

                return carry

            carry = lax.fori_loop(0, nch // _NBUF, dbl_step, carry0)

            ms = carry[:8]
            gs = carry[8:]
            for r in range(8):
                rmax = jnp.max(ms[r])
                cand = jnp.where(ms[r] == rmax, gs[r], jnp.int32(10**9))
                ridx = jnp.min(cand)
                srow = g * 8 + r
                half = srow // nl
                sel = lane == (srow % nl)
                mh = mstage[pl.ds(half * nl, nl)]
                ih = istage[pl.ds(half * nl, nl)]
                mstage[pl.ds(half * nl, nl)] = jnp.where(sel, rmax, mh)
                istage[pl.ds(half * nl, nl)] = jnp.where(sel, ridx, ih)

        pltpu.sync_copy(mstage, max_hbm.at[pl.ds(row0, rows_per_w)])
        pltpu.sync_copy(istage, idx_hbm.at[pl.ds(row0, rows_per_w)])

    return sc_argmax_kernel(inputs)


def _gather_sc(embeddings, tc_max, tc_idx, sc_max, sc_idx):
    (b,) = tc_idx.shape
    v, d = embeddings.shape
    info = plsc.get_sparse_core_info()
    nw = info.num_cores * info.num_subcores
    nl = info.num_lanes
    assert b % (8 * nw) == 0 and d % nl == 0
    b_per_w = b // nw
    mesh = plsc.VectorSubcoreMesh(core_axis_name="c", subcore_axis_name="s", num_cores=2)

    @functools.partial(
        pl.kernel,
        mesh=mesh,
        out_type=jax.ShapeDtypeStruct((b, d), jnp.float32),
        scratch_types=[
            pltpu.VMEM((b_per_w,), jnp.float32),
            pltpu.VMEM((b_per_w,), jnp.int32),
            pltpu.VMEM((b_per_w,), jnp.float32),
            pltpu.VMEM((b_per_w,), jnp.int32),
            pltpu.VMEM((b_per_w,), jnp.int32),
            pltpu.VMEM((b_per_w, d), jnp.float32),
            pltpu.SemaphoreType.DMA,
        ],
        compiler_params=pltpu.CompilerParams(use_tc_tiling_on_sc=False),
    )
    def gather_kernel(
        table_hbm,
        tcm_hbm,
        tci_hbm,
        scm_hbm,
        sci_hbm,
        out_hbm,
        tcm_v,
        tci_v,
        scm_v,
        sci_v,
        idx_v,
        rows_v,
        sem,
    ):
        wid = lax.axis_index("s") * info.num_cores + lax.axis_index("c")
        base = wid * b_per_w
        pltpu.sync_copy(tcm_hbm.at[pl.ds(base, b_per_w)], tcm_v)
        pltpu.sync_copy(tci_hbm.at[pl.ds(base, b_per_w)], tci_v)
        pltpu.sync_copy(scm_hbm.at[pl.ds(base, b_per_w)], scm_v)
        pltpu.sync_copy(sci_hbm.at[pl.ds(base, b_per_w)], sci_v)
        for h in range(b_per_w // nl):
            sl = pl.ds(h * nl, nl)
            tm = tcm_v[sl]
            ti = tci_v[sl]
            sm = scm_v[sl]
            si = sci_v[sl]
            upd = tm > sm
            idx_v[sl] = jnp.where(upd, ti, si)
        pltpu.async_copy(table_hbm.at[idx_v], rows_v, sem).wait()
        pltpu.sync_copy(rows_v, out_hbm.at[pl.ds(base, b_per_w)])

    return gather_kernel(embeddings, tc_max, tc_idx, sc_max, sc_idx)


def kernel(inputs, embeddings):
    b = inputs.shape[0]
    sc_max, sc_idx = _argmax_sc(inputs)
    tc_max, tc_idx = _argmax_tc(inputs)
    return _gather_sc(
        embeddings,
        tc_max.reshape(b),
        tc_idx.reshape(b),
        sc_max,
        sc_idx,
    )


# speedup vs baseline: 1.1227x; 1.1227x over previous
import functools

import jax
import jax.numpy as jnp
from jax import lax
from jax.experimental import pallas as pl
from jax.experimental.pallas import tpu as pltpu
from jax.experimental.pallas import tpu_sc as plsc

_LANES = 128
_NBUF = 2
_BR = 8
_C0 = 88 * 1024
_SC_CH = 2048


def _tc_argmax_body(x_hbm, max_ref, idx_ref, buf, sem):
    b, v = x_hbm.shape
    w = v - _C0
    copy = pltpu.make_async_copy(x_hbm.at[:, pl.ds(_C0, w)], buf, sem)
    copy.start()
    copy.wait()
    x = buf[...]
    m = jnp.max(x, axis=1, keepdims=True)
    ii = lax.broadcasted_iota(jnp.int32, x.shape, 1) + _C0
    cand = jnp.where(x == m, ii, jnp.int32(10**9))
    idx_ref[...] = jnp.min(cand, axis=1, keepdims=True)
    max_ref[...] = m


def _argmax_tc(inputs):
    b, v = inputs.shape
    w = v - _C0
    return pl.pallas_call(
        _tc_argmax_body,
        in_specs=[pl.BlockSpec(memory_space=pl.ANY)],
        out_specs=[
            pl.BlockSpec(memory_space=pltpu.MemorySpace.VMEM),
            pl.BlockSpec(memory_space=pltpu.MemorySpace.VMEM),
        ],
        out_shape=[
            jax.ShapeDtypeStruct((b, 1), jnp.float32),
            jax.ShapeDtypeStruct((b, 1), jnp.int32),
        ],
        scratch_shapes=[
            pltpu.VMEM((b, w), jnp.float32),
            pltpu.SemaphoreType.DMA,
        ],
        compiler_params=pltpu.CompilerParams(vmem_limit_bytes=100 * 1024 * 1024),
    )(inputs)


def _argmax_sc(inputs):
    b, v = inputs.shape
    info = plsc.get_sparse_core_info()
    nw = info.num_cores * info.num_subcores
    nl = info.num_lanes
    rows_per_w = b // nw
    ngroups = rows_per_w // 8
    nch = _C0 // _SC_CH
    mesh = plsc.VectorSubcoreMesh(core_axis_name="c", subcore_axis_name="s", num_cores=2)

    @functools.partial(
        pl.kernel,
        mesh=mesh,
        out_type=(
            jax.ShapeDtypeStruct((b,), jnp.float32),
            jax.ShapeDtypeStruct((b,), jnp.int32),
        ),
        scratch_types=[
            pltpu.VMEM((_NBUF, 8, _SC_CH), jnp.float32),
            pltpu.VMEM((rows_per_w,), jnp.float32),
            pltpu.VMEM((rows_per_w,), jnp.int32),
            pltpu.SemaphoreType.DMA((_NBUF,)),
        ],
        compiler_params=pltpu.CompilerParams(needs_layout_passes=False),
    )
    def sc_argmax_kernel(x_hbm, max_hbm, idx_hbm, buf, mstage, istage, sems):
        wid = lax.axis_index("s") * info.num_cores + lax.axis_index("c")
        row0 = wid * rows_per_w
        lane = lax.iota(jnp.int32, nl)
        nvec = _SC_CH // nl

        def start(par, grow, ch):
            pltpu.make_async_copy(
                x_hbm.at[pl.ds(grow, 8), pl.ds(ch * _SC_CH, _SC_CH)],
                buf.at[par],
                sems.at[par],
            ).start()

        def wait(par):
            pltpu.make_async_copy(
                x_hbm.at[pl.ds(0, 8), pl.ds(0, _SC_CH)],
                buf.at[par],
                sems.at[par],
            ).wait()

        for g in range(ngroups):
            grow = row0 + g * 8
            start(0, grow, 0)
            start(1, grow, 1)
            neg = jnp.full((nl,), -jnp.inf, jnp.float32)
            zero = jnp.zeros((nl,), jnp.int32)
            carry0 = (neg,) * 8 + (zero,) * 8

            def dbl_step(i, carry):
                for par in range(_NBUF):
                    ch = i * _NBUF + par
                    wait(par)

                    def vec_step(j, carry2):
                        ms = carry2[:8]
                        gs = carry2[8:]
                        giota = lane + (ch * _SC_CH + j * nl)
                        new_m = []
                        new_g = []
                        for r in range(8):
                            chunk = buf[par, r, pl.ds(j * nl, nl)]
                            upd = chunk > ms[r]
                            new_m.append(jnp.where(upd, chunk, ms[r]))
                            new_g.append(jnp.where(upd, giota, gs[r]))
                        return tuple(new_m) + tuple(new_g)

                    carry = lax.fori_loop(0, nvec, vec_step, carry)

                    @pl.when(ch + _NBUF < nch)
                    def _():
                        start(par, grow, ch + _NBUF)

                return carry

            carry = lax.fori_loop(0, nch // _NBUF, dbl_step, carry0)

            ms = carry[:8]
            gs = carry[8:]
            for r in range(8):
                rmax = jnp.max(ms[r])
                cand = jnp.where(ms[r] == rmax, gs[r], jnp.int32(10**9))
                ridx = jnp.min(cand)
                srow = g * 8 + r
                half = srow // nl
                sel = lane == (srow % nl)
                mh = mstage[pl.ds(half * nl, nl)]
                ih = istage[pl.ds(half * nl, nl)]
                mstage[pl.ds(half * nl, nl)] = jnp.where(sel, rmax, mh)
                istage[pl.ds(half * nl, nl)] = jnp.where(sel, ridx, ih)

        pltpu.sync_copy(mstage, max_hbm.at[pl.ds(row0, rows_per_w)])
        pltpu.sync_copy(istage, idx_hbm.at[pl.ds(row0, rows_per_w)])

    return sc_argmax_kernel(inputs)


def _gather_sc(embeddings, tc_max, tc_idx, sc_max, sc_idx):
    (b,) = tc_idx.shape
    v, d = embeddings.shape
    info = plsc.get_sparse_core_info()
    nw = info.num_cores * info.num_subcores
    nl = info.num_lanes
    assert b % (8 * nw) == 0 and d % nl == 0
    b_per_w = b // nw
    mesh = plsc.VectorSubcoreMesh(core_axis_name="c", subcore_axis_name="s", num_cores=2)

    @functools.partial(
        pl.kernel,
        mesh=mesh,
        out_type=jax.ShapeDtypeStruct((b, d), jnp.float32),
        scratch_types=[
            pltpu.VMEM((b_per_w,), jnp.float32),
            pltpu.VMEM((b_per_w,), jnp.int32),
            pltpu.VMEM((b_per_w,), jnp.float32),
            pltpu.VMEM((b_per_w,), jnp.int32),
            pltpu.VMEM((b_per_w,), jnp.int32),
            pltpu.VMEM((b_per_w, d), jnp.float32),
            pltpu.SemaphoreType.DMA,
        ],
        compiler_params=pltpu.CompilerParams(use_tc_tiling_on_sc=False),
    )
    def gather_kernel(
        table_hbm,
        tcm_hbm,
        tci_hbm,
        scm_hbm,
        sci_hbm,
        out_hbm,
        tcm_v,
        tci_v,
        scm_v,
        sci_v,
        idx_v,
        rows_v,
        sem,
    ):
        wid = lax.axis_index("s") * info.num_cores + lax.axis_index("c")
        base = wid * b_per_w
        pltpu.sync_copy(tcm_hbm.at[pl.ds(base, b_per_w)], tcm_v)
        pltpu.sync_copy(tci_hbm.at[pl.ds(base, b_per_w)], tci_v)
        pltpu.sync_copy(scm_hbm.at[pl.ds(base, b_per_w)], scm_v)
        pltpu.sync_copy(sci_hbm.at[pl.ds(base, b_per_w)], sci_v)
        for h in range(b_per_w // nl):
            sl = pl.ds(h * nl, nl)
            tm = tcm_v[sl]
            ti = tci_v[sl]
            sm = scm_v[sl]
            si = sci_v[sl]
            upd = tm > sm
            idx_v[sl] = jnp.where(upd, ti, si)
        pltpu.async_copy(table_hbm.at[idx_v], rows_v, sem).wait()
        pltpu.sync_copy(rows_v, out_hbm.at[pl.ds(base, b_per_w)])

    return gather_kernel(embeddings, tc_max, tc_idx, sc_max, sc_idx)


def kernel(inputs, embeddings):
    b = inputs.shape[0]
    sc_max, sc_idx = _argmax_sc(inputs)
    tc_max, tc_idx = _argmax_tc(inputs)
    return _gather_sc(
        embeddings,
        tc_max.reshape(b),
        tc_idx.reshape(b),
        sc_max,
        sc_idx,
    )


# speedup vs baseline: 1.1763x; 1.0477x over previous
import functools

import jax
import jax.numpy as jnp
from jax import lax
from jax.experimental import pallas as pl
from jax.experimental.pallas import tpu as pltpu
from jax.experimental.pallas import tpu_sc as plsc

_LANES = 128
_NBUF = 2
_BR = 8
_C0 = 44 * 1024
_SC_CH = 2048
_TC_BC = 4096


def _make_tc_body(v):
    nchunks = _TC_BC // _LANES

    def body(x_ref, max_ref, idx_ref, m_s, g_s):
        j = pl.program_id(0)
        nsteps = pl.num_programs(0)
        br = x_ref.shape[0]
        lane = lax.broadcasted_iota(jnp.int32, (br, _LANES), 1)

        @pl.when(j == 0)
        def _():
            m_s[...] = jnp.full((br, _LANES), -jnp.inf, jnp.float32)
            g_s[...] = jnp.zeros((br, _LANES), jnp.int32)

        m = m_s[...]
        g = g_s[...]
        base0 = _C0 + j * _TC_BC
        for k in range(nchunks):
            chunk = x_ref[:, k * _LANES : (k + 1) * _LANES]
            gidx = lane + (base0 + k * _LANES)
            upd = (chunk > m) & (gidx < v)
            m = jnp.where(upd, chunk, m)
            g = jnp.where(upd, gidx, g)
        m_s[...] = m
        g_s[...] = g

        @pl.when(j == nsteps - 1)
        def _():
            rowmax = jnp.max(m, axis=1, keepdims=True)
            cand = jnp.where(m == rowmax, g, jnp.int32(10**9))
            idx_ref[...] = jnp.min(cand, axis=1, keepdims=True)
            max_ref[...] = rowmax

    return body


def _argmax_tc(inputs):
    b, v = inputs.shape
    nsteps = -(-(v - _C0) // _TC_BC)
    off = _C0 // _TC_BC
    return pl.pallas_call(
        _make_tc_body(v),
        grid=(nsteps,),
        in_specs=[pl.BlockSpec((b, _TC_BC), lambda j: (0, j + off))],
        out_specs=[
            pl.BlockSpec((b, 1), lambda j: (0, 0)),
            pl.BlockSpec((b, 1), lambda j: (0, 0)),
        ],
        out_shape=[
            jax.ShapeDtypeStruct((b, 1), jnp.float32),
            jax.ShapeDtypeStruct((b, 1), jnp.int32),
        ],
        scratch_shapes=[
            pltpu.VMEM((b, _LANES), jnp.float32),
            pltpu.VMEM((b, _LANES), jnp.int32),
        ],
        compiler_params=pltpu.CompilerParams(vmem_limit_bytes=100 * 1024 * 1024),
    )(inputs)


def _argmax_sc(inputs):
    b, v = inputs.shape
    info = plsc.get_sparse_core_info()
    nw = info.num_cores * info.num_subcores
    nl = info.num_lanes
    rows_per_w = b // nw
    ngroups = rows_per_w // 8
    nch = _C0 // _SC_CH
    mesh = plsc.VectorSubcoreMesh(core_axis_name="c", subcore_axis_name="s", num_cores=2)

    @functools.partial(
        pl.kernel,
        mesh=mesh,
        out_type=(
            jax.ShapeDtypeStruct((b,), jnp.float32),
            jax.ShapeDtypeStruct((b,), jnp.int32),
        ),
        scratch_types=[
            pltpu.VMEM((_NBUF, 8, _SC_CH), jnp.float32),
            pltpu.VMEM((rows_per_w,), jnp.float32),
            pltpu.VMEM((rows_per_w,), jnp.int32),
            pltpu.SemaphoreType.DMA((_NBUF,)),
        ],
        compiler_params=pltpu.CompilerParams(needs_layout_passes=False),
    )
    def sc_argmax_kernel(x_hbm, max_hbm, idx_hbm, buf, mstage, istage, sems):
        wid = lax.axis_index("s") * info.num_cores + lax.axis_index("c")
        row0 = wid * rows_per_w
        lane = lax.iota(jnp.int32, nl)
        nvec = _SC_CH // nl

        def start(par, grow, ch):
            pltpu.make_async_copy(
                x_hbm.at[pl.ds(grow, 8), pl.ds(ch * _SC_CH, _SC_CH)],
                buf.at[par],
                sems.at[par],
            ).start()

        def wait(par):
            pltpu.make_async_copy(
                x_hbm.at[pl.ds(0, 8), pl.ds(0, _SC_CH)],
                buf.at[par],
                sems.at[par],
            ).wait()

        for g in range(ngroups):
            grow = row0 + g * 8
            start(0, grow, 0)
            start(1, grow, 1)
            neg = jnp.full((nl,), -jnp.inf, jnp.float32)
            zero = jnp.zeros((nl,), jnp.int32)
            carry0 = (neg,) * 8 + (zero,) * 8

            def dbl_step(i, carry):
                for par in range(_NBUF):
                    ch = i * _NBUF + par
                    wait(par)

                    def vec_step(j, carry2):
                        ms = carry2[:8]
                        gs = carry2[8:]
                        giota = lane + (ch * _SC_CH + j * nl)
                        new_m = []
                        new_g = []
                        for r in range(8):
                            chunk = buf[par, r, pl.ds(j * nl, nl)]
                            upd = chunk > ms[r]
                            new_m.append(jnp.where(upd, chunk, ms[r]))
                            new_g.append(jnp.where(upd, giota, gs[r]))
                        return tuple(new_m) + tuple(new_g)

                    carry = lax.fori_loop(0, nvec, vec_step, carry)

                    @pl.when(ch + _NBUF < nch)
                    def _():
                        start(par, grow, ch + _NBUF)

                return carry

            carry = lax.fori_loop(0, nch // _NBUF, dbl_step, carry0)

            ms = carry[:8]
            gs = carry[8:]
            for r in range(8):
                rmax = jnp.max(ms[r])
                cand = jnp.where(ms[r] == rmax, gs[r], jnp.int32(10**9))
                ridx = jnp.min(cand)
                srow = g * 8 + r
                half = srow // nl
                sel = lane == (srow % nl)
                mh = mstage[pl.ds(half * nl, nl)]
                ih = istage[pl.ds(half * nl, nl)]
                mstage[pl.ds(half * nl, nl)] = jnp.where(sel, rmax, mh)
                istage[pl.ds(half * nl, nl)] = jnp.where(sel, ridx, ih)

        pltpu.sync_copy(mstage, max_hbm.at[pl.ds(row0, rows_per_w)])
        pltpu.sync_copy(istage, idx_hbm.at[pl.ds(row0, rows_per_w)])

    return sc_argmax_kernel(inputs)


def _gather_sc(embeddings, tc_max, tc_idx, sc_max, sc_idx):
    (b,) = tc_idx.shape
    v, d = embeddings.shape
    info = plsc.get_sparse_core_info()
    nw = info.num_cores * info.num_subcores
    nl = info.num_lanes
    assert b % (8 * nw) == 0 and d % nl == 0
    b_per_w = b // nw
    mesh = plsc.VectorSubcoreMesh(core_axis_name="c", subcore_axis_name="s", num_cores=2)

    @functools.partial(
        pl.kernel,
        mesh=mesh,
        out_type=jax.ShapeDtypeStruct((b, d), jnp.float32),
        scratch_types=[
            pltpu.VMEM((b_per_w,), jnp.float32),
            pltpu.VMEM((b_per_w,), jnp.int32),
            pltpu.VMEM((b_per_w,), jnp.float32),
            pltpu.VMEM((b_per_w,), jnp.int32),
            pltpu.VMEM((b_per_w,), jnp.int32),
            pltpu.VMEM((b_per_w, d), jnp.float32),
            pltpu.SemaphoreType.DMA,
        ],
        compiler_params=pltpu.CompilerParams(use_tc_tiling_on_sc=False),
    )
    def gather_kernel(
        table_hbm,
        tcm_hbm,
        tci_hbm,
        scm_hbm,
        sci_hbm,
        out_hbm,
        tcm_v,
        tci_v,
        scm_v,
        sci_v,
        idx_v,
        rows_v,
        sem,
    ):
        wid = lax.axis_index("s") * info.num_cores + lax.axis_index("c")
        base = wid * b_per_w
        pltpu.sync_copy(tcm_hbm.at[pl.ds(base, b_per_w)], tcm_v)
        pltpu.sync_copy(tci_hbm.at[pl.ds(base, b_per_w)], tci_v)
        pltpu.sync_copy(scm_hbm.at[pl.ds(base, b_per_w)], scm_v)
        pltpu.sync_copy(sci_hbm.at[pl.ds(base, b_per_w)], sci_v)
        for h in range(b_per_w // nl):
            sl = pl.ds(h * nl, nl)
            tm = tcm_v[sl]
            ti = tci_v[sl]
            sm = scm_v[sl]
            si = sci_v[sl]
            upd = tm > sm
            idx_v[sl] = jnp.where(upd, ti, si)
        pltpu.async_copy(table_hbm.at[idx_v], rows_v, sem).wait()
        pltpu.sync_copy(rows_v, out_hbm.at[pl.ds(base, b_per_w)])

    return gather_kernel(embeddings, tc_max, tc_idx, sc_max, sc_idx)


def kernel(inputs, embeddings):
    b = inputs.shape[0]
    sc_max, sc_idx = _argmax_sc(inputs)
    tc_max, tc_idx = _argmax_tc(inputs)
    return _gather_sc(
        embeddings,
        tc_max.reshape(b),
        tc_idx.reshape(b),
        sc_max,
        sc_idx,
    )
